# baseline (speedup 1.0000x reference)
import functools

import jax
import jax.numpy as jnp
from jax import lax
from jax.experimental import pallas as pl
from jax.experimental.pallas import tpu as pltpu
from jax.experimental.pallas import tpu_sc as plsc

NUM_USERS = 100000
NUM_ITEMS = 100000
EMBED_K = 64
BATCH = 16384

_INFO = plsc.get_sparse_core_info()
_NC, _NS, _L = _INFO.num_cores, _INFO.num_subcores, _INFO.num_lanes
_NW = _NC * _NS
_BPW = BATCH // _NW
_GROUPS = _BPW // _L


def _mf_dot_body(uidx_hbm, iidx_hbm, w_hbm, h_hbm, out_hbm,
                 uidx_v, iidx_v, u_rows, v_rows, out_v, sem_u, sem_v):
    wid = lax.axis_index("s") * _NC + lax.axis_index("c")
    base = wid * _BPW

    pltpu.sync_copy(uidx_hbm.at[pl.ds(base, _BPW)], uidx_v)
    pltpu.sync_copy(iidx_hbm.at[pl.ds(base, _BPW)], iidx_v)

    cp_u = pltpu.async_copy(w_hbm.at[uidx_v], u_rows, sem_u)
    cp_v = pltpu.async_copy(h_hbm.at[iidx_v], v_rows, sem_v)
    cp_u.wait()
    cp_v.wait()

    lane = lax.iota(jnp.int32, _L)

    def group(g, _):
        rows = g * _L + lane
        acc = jnp.zeros((_L,), jnp.float32)
        for k in range(EMBED_K):
            col = jnp.full((_L,), k, jnp.int32)
            uk = plsc.load_gather(u_rows, [rows, col])
            vk = plsc.load_gather(v_rows, [rows, col])
            acc = acc + uk * vk
        out_v[pl.ds(g * _L, _L)] = acc
        return 0

    lax.fori_loop(0, _GROUPS, group, 0)

    pltpu.sync_copy(out_v, out_hbm.at[pl.ds(base, _BPW)])


@jax.jit
def kernel(x, W, H):
    user_idx = x[:, 0].astype(jnp.int32)
    item_idx = x[:, 1].astype(jnp.int32)

    mf = pl.kernel(
        _mf_dot_body,
        out_type=jax.ShapeDtypeStruct((BATCH,), jnp.float32),
        mesh=plsc.VectorSubcoreMesh(core_axis_name="c", subcore_axis_name="s"),
        scratch_types=[
            pltpu.VMEM((_BPW,), jnp.int32),
            pltpu.VMEM((_BPW,), jnp.int32),
            pltpu.VMEM((_BPW, EMBED_K), jnp.float32),
            pltpu.VMEM((_BPW, EMBED_K), jnp.float32),
            pltpu.VMEM((_BPW,), jnp.float32),
            pltpu.SemaphoreType.DMA,
            pltpu.SemaphoreType.DMA,
        ],
        compiler_params=pltpu.CompilerParams(
            needs_layout_passes=False, use_tc_tiling_on_sc=False),
    )
    return mf(user_idx, item_idx, W, H)
